# Initial kernel scaffold; baseline (speedup 1.0000x reference)
import jax
import jax.numpy as jnp
from jax.experimental import pallas as pl


def kernel(u, v, w, n, edge_index, E, E2, W1, b1, W2, b2, Wa, ba, Wq):
    raise NotImplementedError("write your pallas kernel here")


# speedup vs baseline: 20.4977x; 20.4977x over previous
import jax
import jax.numpy as jnp
from jax import lax
from jax.experimental import pallas as pl
from jax.experimental.pallas import tpu as pltpu
from jax.experimental.pallas import tpu_sc as plsc

F32 = jnp.float32
REG_COEF = 1e-4

NN = 50000
D = 64
HD = 32
NE = 800000
CH = 128
NSUB = NE // CH
MB = 8
MBG = 4
NMACRO = NSUB // MB
NTAIL = NSUB - NMACRO * MB
NTILE = 16
CR = 80
NCHUNK = NN // CR

_MESH = plsc.VectorSubcoreMesh(core_axis_name="c", subcore_axis_name="s")


def _striped_chunks(s, fn):
    cnt = (NCHUNK - s + NTILE - 1) // NTILE

    def step(j, carry):
        fn(s + j * NTILE)
        return carry

    lax.fori_loop(0, cnt, step, 0)


def _deg_body(ei, ones_h, zeros_h, out, acc, ones_v, zbuf, idx8):
    c = lax.axis_index("c")
    s = lax.axis_index("s")
    pltpu.sync_copy(ones_h, ones_v)
    pltpu.sync_copy(zeros_h, zbuf)

    _striped_chunks(s, lambda m: pltpu.sync_copy(zbuf, acc.at[pl.ds(m * CR, CR)]))
    plsc.subcore_barrier()

    g = c * NTILE + s
    lo = g * NMACRO // 32
    hi = (g + 1) * NMACRO // 32

    def step(q, carry):
        pltpu.sync_copy(ei.at[1, pl.ds(q * MB, MB)], idx8)
        for jj in range(MB):
            pltpu.sync_copy(ones_v, acc.at[idx8.at[jj]], add=True)
        return carry

    lax.fori_loop(lo, hi, step, 0)

    @pl.when(g == 31)
    def _():
        pltpu.sync_copy(ei.at[1, pl.ds(NMACRO * MB, NTAIL)],
                        idx8.at[pl.ds(0, NTAIL)])
        for jj in range(NTAIL):
            pltpu.sync_copy(ones_v, acc.at[idx8.at[jj]], add=True)

    plsc.subcore_barrier()

    def wchunk(m):
        pltpu.sync_copy(acc.at[pl.ds(m * CR, CR)], zbuf)
        pltpu.sync_copy(zbuf, out.at[c, pl.ds(m * CR, CR)])

    _striped_chunks(s, wchunk)


_deg_call = pl.kernel(
    _deg_body,
    out_type=jax.ShapeDtypeStruct((2, NN, 8), F32),
    mesh=_MESH,
    compiler_params=pltpu.CompilerParams(use_tc_tiling_on_sc=False),
    scratch_types=[
        pltpu.VMEM_SHARED((NN, 8), F32),
        pltpu.VMEM((CH, 8), F32),
        pltpu.VMEM((CR, 8), F32),
        pltpu.VMEM((MB, CH), jnp.int32),
    ],
)


NPAIR = NMACRO // 2
NLEFT = NMACRO - 2 * NPAIR
HB = 2
NH = MB // HB * 2


def _s_body(xa, xb, ei, z32, d2f, a1a, a1b, x2a, x2b, a2a, a2b,
            acc, zbuf, idxA, idxB, bufA, bufB, wbuf, dbuf,
            gsA, gsB, ssA, ssB):
    c = lax.axis_index("c")
    s = lax.axis_index("s")
    pltpu.sync_copy(z32, zbuf)

    _striped_chunks(s, lambda m: pltpu.sync_copy(zbuf, acc.at[pl.ds(m * CR, CR)]))
    plsc.subcore_barrier()

    lo = s * NPAIR // NTILE
    hi = (s + 1) * NPAIR // NTILE

    def edge_loop(xsrc):
        def half(t):
            buf = bufA if t % 2 == 0 else bufB
            gs = gsA if t % 2 == 0 else gsB
            ss = ssA if t % 2 == 0 else ssB
            idx = idxA if t < NH // 2 else idxB
            r0 = (HB * t) % MB
            return buf, gs, ss, idx, r0

        def fire_g(t):
            buf, gs, _, idx, r0 = half(t)
            for j in range(HB):
                pltpu.async_copy(xsrc.at[idx.at[0, r0 + j]],
                                 buf.at[pl.ds(j * CH, CH)], gs)

        def wait_g(t):
            buf, gs, _, idx, r0 = half(t)
            for j in range(HB):
                pltpu.make_async_copy(xsrc.at[idx.at[0, r0 + j]],
                                      buf.at[pl.ds(j * CH, CH)], gs).wait()

        def fire_s(t):
            buf, _, ss, idx, r0 = half(t)
            for j in range(HB):
                pltpu.async_copy(buf.at[pl.ds(j * CH, CH)],
                                 acc.at[idx.at[1, r0 + j]], ss, add=True)

        def wait_s(t):
            buf, _, ss, idx, r0 = half(t)
            for j in range(HB):
                pltpu.make_async_copy(buf.at[pl.ds(j * CH, CH)],
                                      acc.at[idx.at[1, r0 + j]], ss).wait()

        def load_idx(dst, m):
            pltpu.sync_copy(ei.at[pl.ds(0, 2), pl.ds(m * MB, MB)], dst)

        load_idx(idxA, 2 * lo)
        fire_g(0)

        def body(p, carry):
            for t in range(NH):
                if t == 0:
                    @pl.when(p > lo)
                    def _():
                        wait_s(NH - 1)
                    load_idx(idxB, 2 * p + 1)
                if t == NH - 3:
                    @pl.when(p + 1 < hi)
                    def _():
                        load_idx(idxA, 2 * (p + 1))
                if t >= 1:
                    wait_s(t - 1)
                if t + 1 < NH:
                    fire_g(t + 1)
                else:
                    @pl.when(p + 1 < hi)
                    def _():
                        fire_g(0)
                wait_g(t)
                fire_s(t)
            return carry

        lax.fori_loop(lo, hi, body, 0)
        wait_s(NH - 1)

        @pl.when(s == NTILE - 1)
        def _():
            load_idx(idxA, 2 * NPAIR)
            for jj in range(MB):
                pltpu.async_copy(xsrc.at[idxA.at[0, jj]],
                                 bufA.at[pl.ds(0, CH)], gsA).wait()
                pltpu.sync_copy(bufA.at[pl.ds(0, CH)],
                                acc.at[idxA.at[1, jj]], add=True)
            pltpu.sync_copy(ei.at[pl.ds(0, 2), pl.ds(NMACRO * MB, NTAIL)],
                            idxA.at[pl.ds(0, 2), pl.ds(0, NTAIL)])
            for jj in range(NTAIL):
                pltpu.async_copy(xsrc.at[idxA.at[0, jj]],
                                 bufA.at[pl.ds(0, CH)], gsA).wait()
                pltpu.sync_copy(bufA.at[pl.ds(0, CH)],
                                acc.at[idxA.at[1, jj]], add=True)

    def drain_chunk(m, out1, outx):
        r = m * CR
        pltpu.sync_copy(acc.at[pl.ds(r, CR)], wbuf)
        pltpu.sync_copy(zbuf, acc.at[pl.ds(r, CR)])
        pltpu.sync_copy(wbuf, out1.at[pl.ds(r, CR)])
        pltpu.sync_copy(d2f.at[pl.ds(r, CR)], dbuf)

        def scale_row(i, carry):
            for h in range(HD // 16):
                wbuf[i, pl.ds(h * 16, 16)] = (
                    wbuf[i, pl.ds(h * 16, 16)] * dbuf[i, pl.ds(h * 16, 16)])
            return carry

        lax.fori_loop(0, CR, scale_row, 0)
        pltpu.sync_copy(wbuf, outx.at[pl.ds(r, CR)])

    def wb_chunk(m, out2):
        pltpu.sync_copy(acc.at[pl.ds(m * CR, CR)], wbuf)
        pltpu.sync_copy(wbuf, out2.at[pl.ds(m * CR, CR)])

    def core_prog(xsrc, out1, outx, out2):
        edge_loop(xsrc)
        plsc.subcore_barrier()
        _striped_chunks(s, lambda m: drain_chunk(m, out1, outx))
        plsc.subcore_barrier()
        edge_loop(outx)
        plsc.subcore_barrier()
        _striped_chunks(s, lambda m: wb_chunk(m, out2))

    @pl.when(c == 0)
    def _():
        core_prog(xa, a1a, x2a, a2a)

    @pl.when(c == 1)
    def _():
        core_prog(xb, a1b, x2b, a2b)


_s_call = pl.kernel(
    _s_body,
    out_type=tuple(jax.ShapeDtypeStruct((NN, HD), F32) for _ in range(6)),
    mesh=_MESH,
    compiler_params=pltpu.CompilerParams(use_tc_tiling_on_sc=False),
    scratch_types=[
        pltpu.VMEM_SHARED((NN, HD), F32),
        pltpu.VMEM((CR, HD), F32),
        pltpu.VMEM((2, MB, CH), jnp.int32),
        pltpu.VMEM((2, MB, CH), jnp.int32),
        pltpu.VMEM((HB * CH, HD), F32),
        pltpu.VMEM((HB * CH, HD), F32),
        pltpu.VMEM((CR, HD), F32),
        pltpu.VMEM((CR, HD), F32),
        pltpu.SemaphoreType.DMA,
        pltpu.SemaphoreType.DMA,
        pltpu.SemaphoreType.DMA,
        pltpu.SemaphoreType.DMA,
    ],
)


NG = 4096 + 4096 + 4096 * 40
GMACRO = NG // (MB * CH)


def _gather_body(z, gidx, out, idx8, gbuf, sem):
    c = lax.axis_index("c")
    s = lax.axis_index("s")
    w = s * 2 + c
    lo = w * GMACRO // 32
    hi = (w + 1) * GMACRO // 32

    def step(q, carry):
        pltpu.sync_copy(gidx.at[q], idx8)
        cps = [
            pltpu.async_copy(z.at[idx8.at[jj]],
                             gbuf.at[pl.ds(jj * CH, CH)], sem)
            for jj in range(MB)
        ]
        for cp in cps:
            cp.wait()
        pltpu.sync_copy(gbuf, out.at[pl.ds(q * MB * CH, MB * CH)])
        return carry

    lax.fori_loop(lo, hi, step, 0)


_gather_call = pl.kernel(
    _gather_body,
    out_type=jax.ShapeDtypeStruct((NG, D), F32),
    mesh=_MESH,
    compiler_params=pltpu.CompilerParams(use_tc_tiling_on_sc=False),
    scratch_types=[
        pltpu.VMEM((MB, CH), jnp.int32),
        pltpu.VMEM((MB * CH, D), F32),
        pltpu.SemaphoreType.DMA,
    ],
)


RB = 2000


def _prep_body(d0, d1, e, dis_ref, d2f_ref, xa_ref, xb_ref):
    deg = d0[0][:, 0:1] + d1[0][:, 0:1]
    dis = jnp.where(deg > 0, lax.rsqrt(jnp.maximum(deg, 1e-12)), 0.0)
    dis_ref[...] = dis
    d2f_ref[...] = jnp.broadcast_to(dis * dis, (RB, HD))
    x = e[...] * dis
    xa_ref[...] = x[:, :HD]
    xb_ref[...] = x[:, HD:]


_prep_call = pl.pallas_call(
    _prep_body,
    grid=(NN // RB,),
    in_specs=[
        pl.BlockSpec((1, RB, 8), lambda i: (0, i, 0)),
        pl.BlockSpec((1, RB, 8), lambda i: (1, i, 0)),
        pl.BlockSpec((RB, D), lambda i: (i, 0)),
    ],
    out_specs=[
        pl.BlockSpec((RB, 1), lambda i: (i, 0)),
        pl.BlockSpec((RB, HD), lambda i: (i, 0)),
        pl.BlockSpec((RB, HD), lambda i: (i, 0)),
        pl.BlockSpec((RB, HD), lambda i: (i, 0)),
    ],
    out_shape=[
        jax.ShapeDtypeStruct((NN, 1), F32),
        jax.ShapeDtypeStruct((NN, HD), F32),
        jax.ShapeDtypeStruct((NN, HD), F32),
        jax.ShapeDtypeStruct((NN, HD), F32),
    ],
)


def _mm(x, w):
    return lax.dot_general(
        x, w, (((1,), (1,)), ((), ())),
        precision=lax.Precision.DEFAULT,
        preferred_element_type=F32,
    )


def _dense_body(e, e2, a1a, a1b, a2a, a2b, dis, w1, b1, w2, b2, wa, ba, wq, z_ref):
    dis_ = dis[...]
    a1 = jnp.concatenate([a1a[...], a1b[...]], axis=1)
    a2 = jnp.concatenate([a2a[...], a2b[...]], axis=1)
    z_p = (e[...] + dis_ * a1 + dis_ * a2) / 3.0
    h = jax.nn.relu(_mm(e2[...], w1[...]) + b1[...])
    h = jax.nn.relu(_mm(h, w2[...]) + b2[...])
    tp = jnp.tanh(_mm(z_p, wa[...]) + ba[...])
    tn = jnp.tanh(_mm(h, wa[...]) + ba[...])
    wp = _mm(tp, wq[...])
    wn = _mm(tn, wq[...])
    m = jnp.maximum(wp, wn)
    ep = jnp.exp(wp - m)
    en = jnp.exp(wn - m)
    z_ref[...] = (ep * z_p + en * h) / (ep + en)


_dense_call = pl.pallas_call(
    _dense_body,
    grid=(NN // RB,),
    in_specs=[
        pl.BlockSpec((RB, D), lambda i: (i, 0)),
        pl.BlockSpec((RB, D), lambda i: (i, 0)),
        pl.BlockSpec((RB, HD), lambda i: (i, 0)),
        pl.BlockSpec((RB, HD), lambda i: (i, 0)),
        pl.BlockSpec((RB, HD), lambda i: (i, 0)),
        pl.BlockSpec((RB, HD), lambda i: (i, 0)),
        pl.BlockSpec((RB, 1), lambda i: (i, 0)),
        pl.BlockSpec((D, D), lambda i: (0, 0)),
        pl.BlockSpec((1, D), lambda i: (0, 0)),
        pl.BlockSpec((D, D), lambda i: (0, 0)),
        pl.BlockSpec((1, D), lambda i: (0, 0)),
        pl.BlockSpec((D, D), lambda i: (0, 0)),
        pl.BlockSpec((1, D), lambda i: (0, 0)),
        pl.BlockSpec((1, D), lambda i: (0, 0)),
    ],
    out_specs=pl.BlockSpec((RB, D), lambda i: (i, 0)),
    out_shape=jax.ShapeDtypeStruct((NN, D), F32),
)


BB = 512
K = 40


def _loss_body(*refs):
    gu_ = refs[0][...]
    gv_ = refs[1][...]
    w_ = refs[2][...]
    gns = refs[3:3 + K]
    out_ref = refs[3 + K]
    ones_col = jnp.ones((D, 1), F32)

    def rowsum(x):
        return lax.dot_general(x, ones_col, (((1,), (0,)), ((), ())),
                               precision=lax.Precision.DEFAULT,
                               preferred_element_type=F32)

    ps = rowsum(gu_ * gv_)
    coef = (-0.5) * jnp.sign(w_) + 1.5
    x0 = coef * ps
    sq = gu_ * gu_ + gv_ * gv_
    lsacc = jnp.zeros((BB, 1), F32)
    for k in range(K):
        gn_ = gns[k][...]
        x = x0 - rowsum(gu_ * gn_)
        lsacc += jnp.minimum(x, 0.0) - jnp.log1p(jnp.exp(-jnp.abs(x)))
        sq += gn_ * gn_
    val = -jnp.sum(lsacc) + REG_COEF * jnp.sum(rowsum(sq))

    @pl.when(pl.program_id(0) == 0)
    def _():
        out_ref[...] = jnp.zeros_like(out_ref)

    out_ref[...] += val


def _gnk_spec(k):
    return pl.BlockSpec((BB, D), lambda i, k=k: (16 + 8 * k + i, 0))


_loss_call = pl.pallas_call(
    _loss_body,
    grid=(4096 // BB,),
    in_specs=[
        pl.BlockSpec((BB, D), lambda i: (i, 0)),
        pl.BlockSpec((BB, D), lambda i: (8 + i, 0)),
        pl.BlockSpec((BB, 1), lambda i: (i, 0)),
    ] + [_gnk_spec(k) for k in range(K)],
    out_specs=pl.BlockSpec((1, 1), lambda i: (0, 0)),
    out_shape=jax.ShapeDtypeStruct((1, 1), F32),
)


def kernel(u, v, w, n, edge_index, E, E2, W1, b1, W2, b2, Wa, ba, Wq):
    ei3 = edge_index.astype(jnp.int32).reshape(2, NSUB, CH)
    ones8 = jnp.ones((CH, 8), F32)
    zeros8 = jnp.zeros((CR, 8), F32)
    zeros32 = jnp.zeros((CR, HD), F32)

    degp = _deg_call(ei3, ones8, zeros8)
    dis, d2f, xa, xb = _prep_call(degp, degp, E)
    a1a, a1b, _x2a, _x2b, a2a, a2b = _s_call(xa, xb, ei3, zeros32, d2f)
    Z = _dense_call(
        E, E2, a1a, a1b, a2a, a2b, dis,
        W1, b1.reshape(1, D), W2, b2.reshape(1, D),
        Wa, ba.reshape(1, D), Wq.reshape(1, D),
    )
    gidx = jnp.concatenate(
        [u.astype(jnp.int32), v.astype(jnp.int32),
         n.astype(jnp.int32).T.reshape(-1)]
    ).reshape(GMACRO, MB, CH)
    G = _gather_call(Z, gidx)
    loss = _loss_call(G, G, w.reshape(4096, 1), *([G] * K))
    return loss.reshape(())
